# Initial kernel scaffold; baseline (speedup 1.0000x reference)
import jax
import jax.numpy as jnp
from jax.experimental import pallas as pl


def kernel(x, emb):
    raise NotImplementedError("write your pallas kernel here")


# speedup vs baseline: 2.5253x; 2.5253x over previous
import jax
import jax.numpy as jnp
from jax.experimental import pallas as pl


def _copy_block(emb_ref, o_ref):
    o_ref[...] = emb_ref[...]


def kernel(x, emb):
    seq = x.shape[1]
    d = emb.shape[1]
    block = 512
    return pl.pallas_call(
        _copy_block,
        grid=(seq // block,),
        in_specs=[pl.BlockSpec((block, d), lambda i: (i, 0))],
        out_specs=pl.BlockSpec((block, d), lambda i: (i, 0)),
        out_shape=jax.ShapeDtypeStruct((seq, d), emb.dtype),
    )(emb)
